# Initial kernel scaffold; baseline (speedup 1.0000x reference)
import jax
import jax.numpy as jnp
from jax.experimental import pallas as pl


def kernel(pc1, pc2):
    raise NotImplementedError("write your pallas kernel here")


# speedup vs baseline: 455.9851x; 455.9851x over previous
import functools

import jax
import jax.numpy as jnp
from jax.experimental import pallas as pl

_P = 8192
_BM = 128
_NB = _P // _BM
_D = 8
_THD = 2.0


def _chamfer_kernel(x_ref, yt_ref, out_ref):
    yt = yt_ref[...]
    y2 = jnp.sum(yt * yt, axis=0, keepdims=True)

    def body(i, carry):
        chamx_sum, colmin = carry
        xb = x_ref[pl.ds(i * _BM, _BM), :]
        x2 = jnp.sum(xb * xb, axis=1, keepdims=True)
        cross = jnp.dot(xb, yt, preferred_element_type=jnp.float32)
        d = (x2 + y2) - 2.0 * cross
        rowmin = jnp.min(d, axis=1, keepdims=True)
        rowmin = jnp.where(rowmin >= _THD, 0.0, rowmin)
        chamx_sum = chamx_sum + jnp.sum(rowmin)
        colmin = jnp.minimum(colmin, jnp.min(d, axis=0, keepdims=True))
        return chamx_sum, colmin

    init = (jnp.float32(0.0), jnp.full((1, _P), jnp.inf, dtype=jnp.float32))
    chamx_sum, colmin = jax.lax.fori_loop(0, _NB, body, init)
    colmin = jnp.where(colmin >= _THD, 0.0, colmin)
    chamy_sum = jnp.sum(colmin)
    out_ref[...] = (chamx_sum / _P + chamy_sum / _P).reshape(1, 1)


@jax.jit
def kernel(pc1, pc2):
    x = pc1[0]
    y = pc2[0]
    xp = jnp.pad(x, ((0, 0), (0, _D - 3)))
    ytp = jnp.pad(y, ((0, 0), (0, _D - 3))).T
    out = pl.pallas_call(
        _chamfer_kernel,
        out_shape=jax.ShapeDtypeStruct((1, 1), jnp.float32),
    )(xp, ytp)
    return out[0, 0]


# speedup vs baseline: 541.7898x; 1.1882x over previous
import functools

import jax
import jax.numpy as jnp
from jax.experimental import pallas as pl

_P = 8192
_BM = 256
_NB = _P // _BM
_D = 8
_THD = 2.0


def _chamfer_kernel(x_ref, yt_ref, out_ref):
    yt = yt_ref[...]
    y2 = jnp.sum(yt * yt, axis=0, keepdims=True)

    def body(i, carry):
        chamx_sum, colmin = carry
        xb = x_ref[pl.ds(i * _BM, _BM), :]
        x2 = jnp.sum(xb * xb, axis=1, keepdims=True)
        cross = jnp.dot(xb, yt, preferred_element_type=jnp.float32)
        d = (x2 + y2) - 2.0 * cross
        rowmin = jnp.min(d, axis=1, keepdims=True)
        rowmin = jnp.where(rowmin >= _THD, 0.0, rowmin)
        chamx_sum = chamx_sum + jnp.sum(rowmin)
        colmin = jnp.minimum(colmin, jnp.min(d, axis=0, keepdims=True))
        return chamx_sum, colmin

    init = (jnp.float32(0.0), jnp.full((1, _P), jnp.inf, dtype=jnp.float32))
    chamx_sum, colmin = jax.lax.fori_loop(0, _NB, body, init)
    colmin = jnp.where(colmin >= _THD, 0.0, colmin)
    chamy_sum = jnp.sum(colmin)
    out_ref[...] = (chamx_sum / _P + chamy_sum / _P).reshape(1, 1)


@jax.jit
def kernel(pc1, pc2):
    x = pc1[0]
    y = pc2[0]
    xp = jnp.pad(x, ((0, 0), (0, _D - 3)))
    ytp = jnp.pad(y, ((0, 0), (0, _D - 3))).T
    out = pl.pallas_call(
        _chamfer_kernel,
        out_shape=jax.ShapeDtypeStruct((1, 1), jnp.float32),
    )(xp, ytp)
    return out[0, 0]
